# baseline (speedup 1.0000x reference)
import functools

import jax
import jax.numpy as jnp
from jax import lax
from jax.experimental import pallas as pl
from jax.experimental.pallas import tpu as pltpu
from jax.experimental.pallas import tpu_sc as plsc

_P = 0.1
_SEED = 42
_LANES = 16
_CHUNK = 128
_PASS = 256
_WIN = 704
_EOSB = 64


@functools.lru_cache(maxsize=None)
def _compaction_kernel(B: int, L: int, V: int):
    mesh = plsc.VectorSubcoreMesh(core_axis_name="c", subcore_axis_name="s")
    n_vecs = L // _LANES
    half = L // 2
    n_pass = half // _PASS
    n_chunks_p = _PASS // _CHUNK
    n_groups = _CHUNK // _LANES

    @functools.partial(
        pl.kernel,
        mesh=mesh,
        out_type=jax.ShapeDtypeStruct((B, L, V), jnp.float32),
        scratch_types=[
            pltpu.VMEM((L,), jnp.int32),
            pltpu.VMEM((_WIN, V), jnp.float32),
            pltpu.VMEM((_CHUNK, V), jnp.float32),
            pltpu.VMEM((_EOSB, V), jnp.float32),
            pltpu.SemaphoreType.DMA,
        ],
        compiler_params=pltpu.CompilerParams(needs_layout_passes=False),
    )
    def kern(msg_hbm, keep_hbm, eos_hbm, out_hbm,
             src_v, win_v, buf_v, eos_v, stage_sem):
        cid = lax.axis_index("c")
        sid = lax.axis_index("s")
        wid = sid * 2 + cid
        b = wid // 2
        h = wid % 2
        hbase = h * half

        stage = pltpu.async_copy(
            msg_hbm.at[b, pl.ds(hbase, _WIN)], win_v, stage_sem)

        pltpu.sync_copy(keep_hbm.at[pl.ds(b * L, L)], src_v)
        pltpu.sync_copy(eos_hbm, eos_v)

        lane = lax.iota(jnp.int32, _LANES)

        def scan_step(i, cnt):
            kv = src_v[pl.ds(i * _LANES, _LANES)]
            s = jnp.cumsum(kv)
            slots = s + (cnt - 1)
            pos = i * _LANES + lane
            plsc.store_scatter(src_v, [slots], pos, mask=kv > 0)
            return cnt + jnp.max(s)

        num_kept = lax.fori_loop(0, n_vecs, scan_step, jnp.int32(0))

        def fill_tail(i, c):
            vec = i * _LANES + lane
            plsc.store_scatter(src_v, [vec], vec, mask=vec >= num_kept)
            return c

        lax.fori_loop(num_kept // _LANES, n_vecs, fill_tail, 0)

        stage.wait()

        eos_head = jnp.where(lane == 0, 1.0, 0.0).astype(jnp.float32)
        eos_zero = jnp.zeros((_LANES,), jnp.float32)

        def gather_chunk(S, wb):
            def group(g, c):
                rows = src_v[pl.ds(S + g * _LANES, _LANES)] - wb
                out_rows = g * _LANES + lane

                def col(t, c2):
                    tv = jnp.zeros((_LANES,), jnp.int32) + t
                    vals = plsc.load_gather(win_v, [rows, tv])
                    plsc.store_scatter(buf_v, [out_rows, tv], vals)
                    return c2

                lax.fori_loop(0, V, col, 0)
                return c

            lax.fori_loop(0, n_groups, group, 0)

        def run_pass(q, c):
            qbase = hbase + q * _PASS
            wb = jnp.minimum(qbase, L - _WIN)

            @pl.when(q > 0)
            def _stage():
                pltpu.sync_copy(msg_hbm.at[b, pl.ds(wb, _WIN)], win_v)

            r = jnp.clip(num_kept - qbase, 0, _PASS)
            nf = r // _CHUNK
            c0 = r - nf * _CHUNK

            def gather_step(j, c2):
                S = qbase + j * _CHUNK
                gather_chunk(S, wb)
                pltpu.sync_copy(buf_v, out_hbm.at[b, pl.ds(S, _CHUNK)])
                return c2

            lax.fori_loop(0, nf, gather_step, 0)

            @pl.when(c0 > 0)
            def _boundary():
                S = qbase + nf * _CHUNK
                gather_chunk(S, wb)

                def fix(j, c2):
                    buf_v[j, pl.ds(0, _LANES)] = eos_head
                    for k in range(1, V // _LANES):
                        buf_v[j, pl.ds(k * _LANES, _LANES)] = eos_zero
                    return c2

                lax.fori_loop(c0, _CHUNK, fix, 0)
                pltpu.sync_copy(buf_v, out_hbm.at[b, pl.ds(S, _CHUNK)])

            pad0 = (nf + jnp.where(c0 > 0, 1, 0)) * (_CHUNK // _EOSB)

            def pad_step(j, c2):
                pltpu.sync_copy(
                    eos_v, out_hbm.at[b, pl.ds(qbase + j * _EOSB, _EOSB)])
                return c2

            lax.fori_loop(pad0, _PASS // _EOSB, pad_step, 0)
            return c

        lax.fori_loop(0, n_pass, run_pass, 0)

    return kern


def kernel(message, message_length, apply_noise):
    B, L, V = message.shape
    target = jax.random.uniform(jax.random.key(_SEED), (B, L)) < _P
    not_eosed = jnp.arange(L)[None, :] < (message_length - 1)[:, None]
    delete = jnp.logical_and(target, not_eosed)
    delete = jnp.logical_and(delete, jnp.asarray(apply_noise) != 0)
    keep = (1 - delete.astype(jnp.int32)).reshape(B * L)
    eos = jnp.zeros((_EOSB, V), jnp.float32).at[:, 0].set(1.0)
    return _compaction_kernel(B, L, V)(message, keep, eos)


# speedup vs baseline: 1.0018x; 1.0018x over previous
import functools

import jax
import jax.numpy as jnp
from jax import lax
from jax.experimental import pallas as pl
from jax.experimental.pallas import tpu as pltpu
from jax.experimental.pallas import tpu_sc as plsc

_P = 0.1
_SEED = 42
_LANES = 16
_CHUNK = 128
_PASS = 256
_WIN = 704
_EOSB = 64


@functools.lru_cache(maxsize=None)
def _compaction_kernel(B: int, L: int, V: int):
    mesh = plsc.VectorSubcoreMesh(core_axis_name="c", subcore_axis_name="s")
    n_vecs = L // _LANES
    half = L // 2
    n_pass = half // _PASS
    n_chunks_p = _PASS // _CHUNK
    n_groups = _CHUNK // _LANES

    @functools.partial(
        pl.kernel,
        mesh=mesh,
        out_type=jax.ShapeDtypeStruct((B, L, V), jnp.float32),
        scratch_types=[
            pltpu.VMEM((L,), jnp.int32),
            pltpu.VMEM((_WIN, V), jnp.float32),
            pltpu.VMEM((_CHUNK, V), jnp.float32),
            pltpu.VMEM((_EOSB, V), jnp.float32),
            pltpu.SemaphoreType.DMA,
        ],
        compiler_params=pltpu.CompilerParams(needs_layout_passes=False),
    )
    def kern(msg_hbm, keep_hbm, eos_hbm, out_hbm,
             src_v, win_v, buf_v, eos_v, stage_sem):
        cid = lax.axis_index("c")
        sid = lax.axis_index("s")
        wid = sid * 2 + cid
        b = wid // 2
        h = wid % 2
        hbase = h * half

        stage = pltpu.async_copy(
            msg_hbm.at[b, pl.ds(hbase, _WIN)], win_v, stage_sem)

        pltpu.sync_copy(keep_hbm.at[pl.ds(b * L, L)], src_v)
        pltpu.sync_copy(eos_hbm, eos_v)

        lane = lax.iota(jnp.int32, _LANES)

        def scan_step(i, cnt):
            kv = src_v[pl.ds(i * _LANES, _LANES)]
            s = jnp.cumsum(kv)
            slots = s + (cnt - 1)
            pos = i * _LANES + lane
            plsc.store_scatter(src_v, [slots], pos, mask=kv > 0)
            return cnt + jnp.max(s)

        num_kept = lax.fori_loop(0, n_vecs, scan_step, jnp.int32(0))

        def fill_tail(i, c):
            vec = i * _LANES + lane
            plsc.store_scatter(src_v, [vec], vec, mask=vec >= num_kept)
            return c

        lax.fori_loop(num_kept // _LANES, n_vecs, fill_tail, 0)

        stage.wait()

        eos_head = jnp.where(lane == 0, 1.0, 0.0).astype(jnp.float32)
        eos_zero = jnp.zeros((_LANES,), jnp.float32)

        def gather_chunk(S, wb):
            def group(g, c):
                rows = src_v[pl.ds(S + g * _LANES, _LANES)] - wb
                out_rows = g * _LANES + lane

                for t in range(V):
                    tv = jnp.full((_LANES,), t, jnp.int32)
                    vals = plsc.load_gather(win_v, [rows, tv])
                    plsc.store_scatter(buf_v, [out_rows, tv], vals)
                return c

            lax.fori_loop(0, n_groups, group, 0)

        def run_pass(q, c):
            qbase = hbase + q * _PASS
            wb = jnp.minimum(qbase, L - _WIN)

            @pl.when(q > 0)
            def _stage():
                pltpu.sync_copy(msg_hbm.at[b, pl.ds(wb, _WIN)], win_v)

            r = jnp.clip(num_kept - qbase, 0, _PASS)
            nf = r // _CHUNK
            c0 = r - nf * _CHUNK

            def gather_step(j, c2):
                S = qbase + j * _CHUNK
                gather_chunk(S, wb)
                pltpu.sync_copy(buf_v, out_hbm.at[b, pl.ds(S, _CHUNK)])
                return c2

            lax.fori_loop(0, nf, gather_step, 0)

            @pl.when(c0 > 0)
            def _boundary():
                S = qbase + nf * _CHUNK
                gather_chunk(S, wb)

                def fix(j, c2):
                    buf_v[j, pl.ds(0, _LANES)] = eos_head
                    for k in range(1, V // _LANES):
                        buf_v[j, pl.ds(k * _LANES, _LANES)] = eos_zero
                    return c2

                lax.fori_loop(c0, _CHUNK, fix, 0)
                pltpu.sync_copy(buf_v, out_hbm.at[b, pl.ds(S, _CHUNK)])

            pad0 = (nf + jnp.where(c0 > 0, 1, 0)) * (_CHUNK // _EOSB)

            def pad_step(j, c2):
                pltpu.sync_copy(
                    eos_v, out_hbm.at[b, pl.ds(qbase + j * _EOSB, _EOSB)])
                return c2

            lax.fori_loop(pad0, _PASS // _EOSB, pad_step, 0)
            return c

        lax.fori_loop(0, n_pass, run_pass, 0)

    return kern


def kernel(message, message_length, apply_noise):
    B, L, V = message.shape
    target = jax.random.uniform(jax.random.key(_SEED), (B, L)) < _P
    not_eosed = jnp.arange(L)[None, :] < (message_length - 1)[:, None]
    delete = jnp.logical_and(target, not_eosed)
    delete = jnp.logical_and(delete, jnp.asarray(apply_noise) != 0)
    keep = (1 - delete.astype(jnp.int32)).reshape(B * L)
    eos = jnp.zeros((_EOSB, V), jnp.float32).at[:, 0].set(1.0)
    return _compaction_kernel(B, L, V)(message, keep, eos)


# speedup vs baseline: 1.6977x; 1.6946x over previous
import functools

import jax
import jax.numpy as jnp
from jax import lax
from jax.experimental import pallas as pl
from jax.experimental.pallas import tpu as pltpu
from jax.experimental.pallas import tpu_sc as plsc

_P = 0.1
_SEED = 42
_LANES = 16
_CHUNK = 128


@functools.lru_cache(maxsize=None)
def _compaction_kernel(B: int, L: int, V: int):
    mesh = plsc.VectorSubcoreMesh(core_axis_name="c", subcore_axis_name="s")
    n_vecs = L // _LANES

    @functools.partial(
        pl.kernel,
        mesh=mesh,
        out_type=jax.ShapeDtypeStruct((B * L, V), jnp.float32),
        scratch_types=[
            pltpu.VMEM((L,), jnp.int32),
            pltpu.VMEM((L,), jnp.int32),
            pltpu.VMEM((_CHUNK, V), jnp.float32),
            pltpu.VMEM((_CHUNK, V), jnp.float32),
            pltpu.SemaphoreType.DMA,
        ],
        compiler_params=pltpu.CompilerParams(
            needs_layout_passes=False, use_tc_tiling_on_sc=False),
    )
    def kern(msg_hbm, keep_hbm, eos_hbm, out_hbm, keep_v, src_v, buf_v, eos_v, sem):
        cid = lax.axis_index("c")
        sid = lax.axis_index("s")
        wid = sid * 2 + cid

        @pl.when(wid < B)
        def _():
            b = wid
            base = b * L
            pltpu.sync_copy(keep_hbm.at[b], keep_v)
            pltpu.sync_copy(eos_hbm, eos_v)

            def fill(i, c):
                src_v[pl.ds(i * _LANES, _LANES)] = jnp.full(
                    (_LANES,), base, jnp.int32)
                return c

            lax.fori_loop(0, n_vecs, fill, 0)

            def scan_step(i, cnt):
                kv = keep_v[pl.ds(i * _LANES, _LANES)]
                s = jnp.cumsum(kv)
                slots = s + (cnt - 1)
                pos = base + i * _LANES + lax.iota(jnp.int32, _LANES)
                plsc.store_scatter(src_v, [slots], pos, mask=kv > 0)
                return cnt + jnp.max(s)

            num_kept = lax.fori_loop(0, n_vecs, scan_step, jnp.int32(0))

            n_full = num_kept // _CHUNK

            def gather_step(i, c):
                idx = src_v.at[pl.ds(i * _CHUNK, _CHUNK)]
                pltpu.async_copy(msg_hbm.at[idx], buf_v, sem).wait()
                pltpu.sync_copy(
                    buf_v, out_hbm.at[pl.ds(base + i * _CHUNK, _CHUNK)])
                return c

            lax.fori_loop(0, n_full, gather_step, 0)

            c0 = num_kept - n_full * _CHUNK
            eos_head = jnp.where(
                lax.iota(jnp.int32, _LANES) == 0, 1.0, 0.0
            ).astype(jnp.float32)
            eos_zero = jnp.zeros((_LANES,), jnp.float32)

            @pl.when(c0 > 0)
            def _boundary():
                idx = src_v.at[pl.ds(n_full * _CHUNK, _CHUNK)]
                pltpu.async_copy(msg_hbm.at[idx], buf_v, sem).wait()

                def fix(j, c):
                    buf_v[j, pl.ds(0, _LANES)] = eos_head
                    for k in range(1, V // _LANES):
                        buf_v[j, pl.ds(k * _LANES, _LANES)] = eos_zero
                    return c

                lax.fori_loop(c0, _CHUNK, fix, 0)
                pltpu.sync_copy(
                    buf_v, out_hbm.at[pl.ds(base + n_full * _CHUNK, _CHUNK)])

            pad0 = n_full + jnp.where(c0 > 0, 1, 0)

            def pad_step(i, c):
                pltpu.sync_copy(eos_v, out_hbm.at[pl.ds(base + i * _CHUNK, _CHUNK)])
                return c

            lax.fori_loop(pad0, L // _CHUNK, pad_step, 0)

    return kern


def kernel(message, message_length, apply_noise):
    B, L, V = message.shape
    target = jax.random.uniform(jax.random.key(_SEED), (B, L)) < _P
    not_eosed = jnp.arange(L)[None, :] < (message_length - 1)[:, None]
    delete = jnp.logical_and(target, not_eosed)
    delete = jnp.logical_and(delete, jnp.asarray(apply_noise) != 0)
    keep = 1 - delete.astype(jnp.int32)
    eos = jnp.zeros((_CHUNK, V), jnp.float32).at[:, 0].set(1.0)
    msg_flat = message.reshape(B * L, V)
    out = _compaction_kernel(B, L, V)(msg_flat, keep, eos)
    return out.reshape(B, L, V)


# speedup vs baseline: 1.9460x; 1.1463x over previous
import functools

import jax
import jax.numpy as jnp
from jax import lax
from jax.experimental import pallas as pl
from jax.experimental.pallas import tpu as pltpu
from jax.experimental.pallas import tpu_sc as plsc

_P = 0.1
_SEED = 42
_LANES = 16
_CHUNK = 128


@functools.lru_cache(maxsize=None)
def _compaction_kernel(B: int, L: int, V: int):
    mesh = plsc.VectorSubcoreMesh(core_axis_name="c", subcore_axis_name="s")
    n_vecs = L // _LANES

    @functools.partial(
        pl.kernel,
        mesh=mesh,
        out_type=jax.ShapeDtypeStruct((B * L, V), jnp.float32),
        scratch_types=[
            pltpu.VMEM((L,), jnp.int32),
            pltpu.VMEM((L,), jnp.int32),
            pltpu.VMEM((_CHUNK, V), jnp.float32),
            pltpu.VMEM((_CHUNK, V), jnp.float32),
            pltpu.SemaphoreType.DMA,
        ],
        compiler_params=pltpu.CompilerParams(
            needs_layout_passes=False, use_tc_tiling_on_sc=False),
    )
    def kern(msg_hbm, keep_hbm, eos_hbm, out_hbm, keep_v, src_v, buf_v, eos_v, sem):
        cid = lax.axis_index("c")
        sid = lax.axis_index("s")
        wid = sid * 2 + cid

        if True:
            b = wid // 2
            h = wid % 2
            base = b * L
            pltpu.sync_copy(keep_hbm.at[b], keep_v)
            pltpu.sync_copy(eos_hbm, eos_v)

            def fill(i, c):
                src_v[pl.ds(i * _LANES, _LANES)] = jnp.full(
                    (_LANES,), base, jnp.int32)
                return c

            lax.fori_loop(0, n_vecs, fill, 0)

            def scan_step(i, cnt):
                kv = keep_v[pl.ds(i * _LANES, _LANES)]
                s = jnp.cumsum(kv)
                slots = s + (cnt - 1)
                pos = base + i * _LANES + lax.iota(jnp.int32, _LANES)
                plsc.store_scatter(src_v, [slots], pos, mask=kv > 0)
                return cnt + jnp.max(s)

            num_kept = lax.fori_loop(0, n_vecs, scan_step, jnp.int32(0))

            n_full = num_kept // _CHUNK

            def gather_step(i, c):
                j = h + 2 * i
                idx = src_v.at[pl.ds(j * _CHUNK, _CHUNK)]
                pltpu.async_copy(msg_hbm.at[idx], buf_v, sem).wait()
                pltpu.sync_copy(
                    buf_v, out_hbm.at[pl.ds(base + j * _CHUNK, _CHUNK)])
                return c

            lax.fori_loop(0, (n_full - h + 1) // 2, gather_step, 0)

            c0 = num_kept - n_full * _CHUNK
            eos_head = jnp.where(
                lax.iota(jnp.int32, _LANES) == 0, 1.0, 0.0
            ).astype(jnp.float32)
            eos_zero = jnp.zeros((_LANES,), jnp.float32)

            @pl.when(jnp.logical_and(c0 > 0, (n_full % 2) == h))
            def _boundary():
                idx = src_v.at[pl.ds(n_full * _CHUNK, _CHUNK)]
                pltpu.async_copy(msg_hbm.at[idx], buf_v, sem).wait()

                def fix(j, c):
                    buf_v[j, pl.ds(0, _LANES)] = eos_head
                    for k in range(1, V // _LANES):
                        buf_v[j, pl.ds(k * _LANES, _LANES)] = eos_zero
                    return c

                lax.fori_loop(c0, _CHUNK, fix, 0)
                pltpu.sync_copy(
                    buf_v, out_hbm.at[pl.ds(base + n_full * _CHUNK, _CHUNK)])

            pad0 = n_full + jnp.where(c0 > 0, 1, 0)
            p0h = pad0 + (pad0 + h) % 2

            def pad_step(i, c):
                j = p0h + 2 * i
                pltpu.sync_copy(eos_v, out_hbm.at[pl.ds(base + j * _CHUNK, _CHUNK)])
                return c

            lax.fori_loop(0, (L // _CHUNK + 1 - p0h) // 2, pad_step, 0)

    return kern


def kernel(message, message_length, apply_noise):
    B, L, V = message.shape
    target = jax.random.uniform(jax.random.key(_SEED), (B, L)) < _P
    not_eosed = jnp.arange(L)[None, :] < (message_length - 1)[:, None]
    delete = jnp.logical_and(target, not_eosed)
    delete = jnp.logical_and(delete, jnp.asarray(apply_noise) != 0)
    keep = 1 - delete.astype(jnp.int32)
    eos = jnp.zeros((_CHUNK, V), jnp.float32).at[:, 0].set(1.0)
    msg_flat = message.reshape(B * L, V)
    out = _compaction_kernel(B, L, V)(msg_flat, keep, eos)
    return out.reshape(B, L, V)


# speedup vs baseline: 1.9877x; 1.0214x over previous
import functools

import jax
import jax.numpy as jnp
from jax import lax
from jax.experimental import pallas as pl
from jax.experimental.pallas import tpu as pltpu
from jax.experimental.pallas import tpu_sc as plsc

_P = 0.1
_SEED = 42
_LANES = 16
_CHUNK = 128


@functools.lru_cache(maxsize=None)
def _compaction_kernel(B: int, L: int, V: int):
    mesh = plsc.VectorSubcoreMesh(core_axis_name="c", subcore_axis_name="s")
    n_vecs = L // _LANES

    @functools.partial(
        pl.kernel,
        mesh=mesh,
        out_type=jax.ShapeDtypeStruct((B * L, V), jnp.float32),
        scratch_types=[
            pltpu.VMEM((L,), jnp.int32),
            pltpu.VMEM((L,), jnp.int32),
            pltpu.VMEM((2, _CHUNK, V), jnp.float32),
            pltpu.VMEM((_CHUNK, V), jnp.float32),
            pltpu.SemaphoreType.DMA,
            pltpu.SemaphoreType.DMA,
            pltpu.SemaphoreType.DMA,
        ],
        compiler_params=pltpu.CompilerParams(
            needs_layout_passes=False, use_tc_tiling_on_sc=False),
    )
    def kern(msg_hbm, keep_hbm, eos_hbm, out_hbm,
             keep_v, src_v, buf_v, eos_v, sem, o0, o1):
        cid = lax.axis_index("c")
        sid = lax.axis_index("s")
        wid = sid * 2 + cid

        if True:
            b = wid // 2
            h = wid % 2
            base = b * L
            pltpu.sync_copy(keep_hbm.at[b], keep_v)
            pltpu.sync_copy(eos_hbm, eos_v)

            def fill(i, c):
                src_v[pl.ds(i * _LANES, _LANES)] = jnp.full(
                    (_LANES,), base, jnp.int32)
                return c

            lax.fori_loop(0, n_vecs, fill, 0)

            def scan_step(i, cnt):
                kv = keep_v[pl.ds(i * _LANES, _LANES)]
                s = jnp.cumsum(kv)
                slots = s + (cnt - 1)
                pos = base + i * _LANES + lax.iota(jnp.int32, _LANES)
                plsc.store_scatter(src_v, [slots], pos, mask=kv > 0)
                return cnt + jnp.max(s)

            num_kept = lax.fori_loop(0, n_vecs, scan_step, jnp.int32(0))

            n_full = num_kept // _CHUNK
            nh = (n_full - h + 1) // 2

            def wait_out(p):
                @pl.when(p == 0)
                def _():
                    pltpu.make_async_copy(
                        buf_v.at[0], out_hbm.at[pl.ds(base, _CHUNK)],
                        o0).wait()

                @pl.when(p == 1)
                def _():
                    pltpu.make_async_copy(
                        buf_v.at[1], out_hbm.at[pl.ds(base, _CHUNK)],
                        o1).wait()

            def gather_step(i, c):
                j = h + 2 * i
                p = i % 2

                @pl.when(i >= 2)
                def _():
                    wait_out(p)

                idx = src_v.at[pl.ds(j * _CHUNK, _CHUNK)]
                dst = out_hbm.at[pl.ds(base + j * _CHUNK, _CHUNK)]

                @pl.when(p == 0)
                def _():
                    pltpu.async_copy(msg_hbm.at[idx], buf_v.at[0], sem).wait()
                    pltpu.async_copy(buf_v.at[0], dst, o0)

                @pl.when(p == 1)
                def _():
                    pltpu.async_copy(msg_hbm.at[idx], buf_v.at[1], sem).wait()
                    pltpu.async_copy(buf_v.at[1], dst, o1)

                return c

            lax.fori_loop(0, nh, gather_step, 0)

            @pl.when(nh >= 2)
            def _():
                wait_out(nh % 2)

            @pl.when(nh >= 1)
            def _():
                wait_out((nh + 1) % 2)

            c0 = num_kept - n_full * _CHUNK
            eos_head = jnp.where(
                lax.iota(jnp.int32, _LANES) == 0, 1.0, 0.0
            ).astype(jnp.float32)
            eos_zero = jnp.zeros((_LANES,), jnp.float32)

            @pl.when(jnp.logical_and(c0 > 0, (n_full % 2) == h))
            def _boundary():
                idx = src_v.at[pl.ds(n_full * _CHUNK, _CHUNK)]
                pltpu.async_copy(msg_hbm.at[idx], buf_v.at[0], sem).wait()

                def fix(j, c):
                    buf_v[0, j, pl.ds(0, _LANES)] = eos_head
                    for k in range(1, V // _LANES):
                        buf_v[0, j, pl.ds(k * _LANES, _LANES)] = eos_zero
                    return c

                lax.fori_loop(c0, _CHUNK, fix, 0)
                pltpu.sync_copy(
                    buf_v.at[0],
                    out_hbm.at[pl.ds(base + n_full * _CHUNK, _CHUNK)])

            pad0 = n_full + jnp.where(c0 > 0, 1, 0)
            p0h = pad0 + (pad0 + h) % 2

            def pad_step(i, c):
                j = p0h + 2 * i
                pltpu.sync_copy(eos_v, out_hbm.at[pl.ds(base + j * _CHUNK, _CHUNK)])
                return c

            lax.fori_loop(0, (L // _CHUNK + 1 - p0h) // 2, pad_step, 0)

    return kern


def kernel(message, message_length, apply_noise):
    B, L, V = message.shape
    target = jax.random.uniform(jax.random.key(_SEED), (B, L)) < _P
    not_eosed = jnp.arange(L)[None, :] < (message_length - 1)[:, None]
    delete = jnp.logical_and(target, not_eosed)
    delete = jnp.logical_and(delete, jnp.asarray(apply_noise) != 0)
    keep = 1 - delete.astype(jnp.int32)
    eos = jnp.zeros((_CHUNK, V), jnp.float32).at[:, 0].set(1.0)
    msg_flat = message.reshape(B * L, V)
    out = _compaction_kernel(B, L, V)(msg_flat, keep, eos)
    return out.reshape(B, L, V)
